# baseline (speedup 1.0000x reference)
import functools

import jax
import jax.numpy as jnp
import numpy as np
from jax import lax
from jax.experimental import pallas as pl
from jax.experimental.pallas import tpu as pltpu
from jax.experimental.pallas import tpu_sc as plsc

N = 50000
E = 150000
C = 128
H = 4
D = 32

NP = 50688
CHUNK = 12672
TPC = 792
NH = 25344
TPH = 1584
EP = 155648
ACC = 72
XC = 64

SA = EP // 32
BA = 128
SB = EP // 16
BC = 128
BB = 128
R1 = 2000
R2 = 4224

_mesh = plsc.VectorSubcoreMesh(core_axis_name="c", subcore_axis_name="s")
_sc_params = pltpu.CompilerParams(needs_layout_passes=False,
                                  use_tc_tiling_on_sc=False)


def _tc1_body(x_ref, w_ref, b_ref, sat_ref, dat_ref, sels_ref, seld_ref,
              xp_ref, asd_ref, mx_ref):
    i = pl.program_id(0)
    xp = jnp.dot(x_ref[...], w_ref[...], preferred_element_type=jnp.float32)
    xp = xp + b_ref[...]
    xp_ref[...] = xp
    asd = jnp.dot(xp * sat_ref[...], sels_ref[...],
                  preferred_element_type=jnp.float32)
    asd = asd + jnp.dot(xp * dat_ref[...], seld_ref[...],
                        preferred_element_type=jnp.float32)
    asd_ref[...] = asd
    bm = jnp.max(asd, axis=0, keepdims=True)
    bmf = jnp.concatenate([bm, jnp.full((1, 112), -1e30, jnp.float32)], axis=1)
    bmb = jnp.broadcast_to(bmf, (8, 128))

    @pl.when(i == 0)
    def _():
        mx_ref[...] = bmb

    @pl.when(i != 0)
    def _():
        mx_ref[...] = jnp.maximum(mx_ref[...], bmb)


_tc1 = pl.pallas_call(
    _tc1_body,
    grid=(N // R1,),
    in_specs=[
        pl.BlockSpec((R1, C), lambda i: (i, 0)),
        pl.BlockSpec((C, C), lambda i: (0, 0)),
        pl.BlockSpec((1, C), lambda i: (0, 0)),
        pl.BlockSpec((1, C), lambda i: (0, 0)),
        pl.BlockSpec((1, C), lambda i: (0, 0)),
        pl.BlockSpec((C, 16), lambda i: (0, 0)),
        pl.BlockSpec((C, 16), lambda i: (0, 0)),
    ],
    out_specs=[
        pl.BlockSpec((R1, C), lambda i: (i, 0)),
        pl.BlockSpec((R1, 16), lambda i: (i, 0)),
        pl.BlockSpec((8, 128), lambda i: (0, 0)),
    ],
    out_shape=[
        jax.ShapeDtypeStruct((N, C), jnp.float32),
        jax.ShapeDtypeStruct((N, 16), jnp.float32),
        jax.ShapeDtypeStruct((8, 128), jnp.float32),
    ],
)


@functools.partial(
    pl.kernel,
    out_type=jax.ShapeDtypeStruct((EP, 8), jnp.int32),
    mesh=_mesh,
    compiler_params=_sc_params,
    scratch_types=[
        pltpu.VMEM((BA,), jnp.int32),
        pltpu.VMEM((BA,), jnp.int32),
        pltpu.VMEM((BA, 16), jnp.float32),
        pltpu.VMEM((BA, 16), jnp.float32),
        pltpu.VMEM((BA, 8), jnp.int32),
        pltpu.VMEM((1, 128), jnp.float32),
        pltpu.SemaphoreType.DMA,
    ],
)
def _sc_a(asd_hbm, srcp_hbm, dstp_hbm, mx_hbm, e_hbm,
          sidx, didx, bufs, bufd, ebuf, mxv, sem):
    w = lax.axis_index("c") * 16 + lax.axis_index("s")
    base0 = w * SA
    pltpu.sync_copy(mx_hbm.at[pl.ds(0, 1)], mxv)
    mv = mxv[0, pl.ds(0, 16)]
    mm = [mv[h] + mv[4 + h] for h in range(4)]
    big_m = [jnp.where(m > 0, m, 0.2 * m) for m in mm]
    i16 = lax.iota(jnp.int32, 16)
    zi16 = jnp.zeros((16,), jnp.int32)

    def zpad(g, _):
        rows = g * 16 + i16
        plsc.store_scatter(ebuf, [rows, jnp.full((16,), 6, jnp.int32)], zi16)
        plsc.store_scatter(ebuf, [rows, jnp.full((16,), 7, jnp.int32)], zi16)
        return 0

    lax.fori_loop(0, BA // 16, zpad, 0)

    def batch_body(bi, _):
        base = base0 + bi * BA
        pltpu.sync_copy(srcp_hbm.at[pl.ds(base, BA)], sidx)
        pltpu.sync_copy(dstp_hbm.at[pl.ds(base, BA)], didx)
        pltpu.async_copy(asd_hbm.at[sidx], bufs, sem).wait()
        pltpu.async_copy(asd_hbm.at[didx], bufd, sem).wait()

        def group_body(g, _):
            rows = g * 16 + i16
            sl = pl.ds(g * 16, 16)
            eid = base + rows
            live = eid < E
            plsc.store_scatter(ebuf, [rows, jnp.full((16,), 0, jnp.int32)],
                               sidx[sl])
            plsc.store_scatter(ebuf, [rows, jnp.full((16,), 1, jnp.int32)],
                               didx[sl])
            for h in range(4):
                ch_s = jnp.full((16,), h, jnp.int32)
                ch_d = jnp.full((16,), 4 + h, jnp.int32)
                s = plsc.load_gather(bufs, [rows, ch_s])
                dd = plsc.load_gather(bufd, [rows, ch_d])
                al = s + dd
                al = jnp.where(al > 0, al, 0.2 * al)
                ev = jnp.exp(al - big_m[h])
                ev = jnp.where(live, ev, 0.0)
                plsc.store_scatter(ebuf, [rows, jnp.full((16,), 2 + h,
                                                         jnp.int32)],
                                   plsc.bitcast(ev, jnp.int32))
            return 0

        lax.fori_loop(0, BA // 16, group_body, 0)
        pltpu.sync_copy(ebuf, e_hbm.at[pl.ds(base, BA)])
        return 0

    lax.fori_loop(0, SA // BA, batch_body, 0)


@functools.partial(
    pl.kernel,
    out_type=jax.ShapeDtypeStruct((2, NP, ACC), jnp.float32),
    mesh=_mesh,
    compiler_params=_sc_params,
    scratch_types=[
        pltpu.VMEM((2, BB), jnp.int32),
        pltpu.VMEM((2, BB), jnp.int32),
        pltpu.VMEM((2, 2, BB), jnp.float32),
        pltpu.VMEM((2, BB, 8), jnp.int32),
        pltpu.VMEM((2, BB, XC), jnp.float32),
        pltpu.VMEM((2, BB, ACC), jnp.float32),
        pltpu.VMEM((SB + 16,), jnp.int32),
        pltpu.VMEM((SB + 16,), jnp.int32),
        pltpu.VMEM_SHARED((CHUNK, ACC), jnp.float32),
        pltpu.SemaphoreType.DMA,
        pltpu.SemaphoreType.DMA,
        pltpu.SemaphoreType.DMA,
        pltpu.SemaphoreType.DMA,
        pltpu.SemaphoreType.DMA,
        pltpu.SemaphoreType.DMA,
    ],
)
def _sc_b(xp2d_hbm, dstp_hbm, e_hbm, nd_hbm,
          sidx2, dloc2, ebuf2, ebat2, rin2, rout2, ldst, elist, acc,
          se0, se1, sr0, sr1, ss0, ss1):
    c = lax.axis_index("c")
    s = lax.axis_index("s")
    base0 = s * SB
    zero16 = jnp.zeros((16,), jnp.float32)
    sent16 = jnp.full((16,), EP - 1, jnp.int32)
    i16 = lax.iota(jnp.int32, 16)
    nchunks = NP // CHUNK
    slots = [(sidx2.at[0], dloc2.at[0], ebuf2.at[0], ebat2.at[0],
              rin2.at[0], rout2.at[0], se0, sr0, ss0),
             (sidx2.at[1], dloc2.at[1], ebuf2.at[1], ebat2.at[1],
              rin2.at[1], rout2.at[1], se1, sr1, ss1)]

    pltpu.sync_copy(dstp_hbm.at[pl.ds(base0, SB)], ldst.at[pl.ds(0, SB)])

    for pi in range(nchunks):
        lo = pi * CHUNK

        def prefill(g, _):
            elist[pl.ds(g * 16, 16)] = sent16
            return 0

        lax.fori_loop(0, SB // 16, prefill, 0)

        def bin_body(g, cnt):
            eid16 = base0 + g * 16 + i16
            dv = ldst[pl.ds(g * 16, 16)]
            mask = (dv >= lo) & (dv < lo + CHUNK)
            plsc.store_compressed(elist.at[pl.ds(cnt, 16)], eid16, mask=mask)
            pc = plsc.all_reduce_population_count(mask)
            return cnt + pc[0]

        cnt = lax.fori_loop(0, SB // 16, bin_body, 0)

        rout0 = rout2.at[0]

        def zrow(r, _):
            for k in range(4):
                rout0[r, pl.ds(16 * k, 16)] = zero16
            rout0[r, pl.ds(ACC - 16, 16)] = zero16
            return 0

        lax.fori_loop(0, BB, zrow, 0)
        for z in range(TPC // BB):
            pltpu.sync_copy(rout0, acc.at[pl.ds(s * TPC + z * BB, BB)])
        rem = TPC % BB
        pltpu.sync_copy(rout0.at[pl.ds(0, rem)],
                        acc.at[pl.ds(s * TPC + TPC - rem, rem)])
        plsc.subcore_barrier()

        nb = lax.div(cnt + (BB - 1), BB)

        @pl.when(nb > 0)
        def _():
            pltpu.async_copy(e_hbm.at[elist.at[pl.ds(0, BB)]],
                             ebat2.at[0], se0)

        def outer_body(bo, _):
            for u in range(2):
                sidx, dloc, ebuf, ebat, rin, rout, sem_e, sem_r, sem_s = \
                    slots[u]
                _, dloc_o, _, ebat_o, _, rout_o, sem_eo, _, sem_so = \
                    slots[1 - u]
                bi = 2 * bo + u

                @pl.when(bi < nb)
                def _():
                    pltpu.make_async_copy(
                        e_hbm.at[elist.at[pl.ds(bi * BB, BB)]], ebat,
                        sem_e).wait()

                    def group_body(g, _):
                        sl = pl.ds(g * 16, 16)
                        rows = g * 16 + i16
                        sv = plsc.load_gather(
                            ebat, [rows, jnp.full((16,), 0, jnp.int32)])
                        dv = plsc.load_gather(
                            ebat, [rows, jnp.full((16,), 1, jnp.int32)])
                        sidx[sl] = sv * 2 + c
                        dloc[sl] = jnp.maximum(dv - lo, 0)
                        for h in range(2):
                            evi = plsc.load_gather(
                                ebat, [rows, jnp.full((16,), 2 + 2 * c + h,
                                                      jnp.int32)])
                            ebuf[h, sl] = plsc.bitcast(evi, jnp.float32)
                        return 0

                    lax.fori_loop(0, BB // 16, group_body, 0)
                    pltpu.async_copy(xp2d_hbm.at[sidx], rin, sem_r)

                    @pl.when(bi + 1 < nb)
                    def _():
                        pltpu.async_copy(
                            e_hbm.at[elist.at[pl.ds((bi + 1) * BB, BB)]],
                            ebat_o, sem_eo)

                    pltpu.make_async_copy(xp2d_hbm.at[sidx], rin,
                                          sem_r).wait()

                    def scale_body(g, _):
                        sl = pl.ds(g * 16, 16)
                        rows = g * 16 + i16
                        evs = []
                        for h in range(2):
                            ev = ebuf[h, sl]
                            evs.append(ev)
                            plsc.store_scatter(
                                rout, [rows, jnp.full((16,), XC + h,
                                                      jnp.int32)], ev)
                        for col in range(XC):
                            cc = jnp.full((16,), col, jnp.int32)
                            vals = plsc.load_gather(rin, [rows, cc])
                            plsc.store_scatter(rout, [rows, cc],
                                               vals * evs[col // 32])
                        return 0

                    lax.fori_loop(0, BB // 16, scale_body, 0)
                    pltpu.async_copy(rout, acc.at[dloc], sem_s,
                                     add=True).wait()
            return 0

        lax.fori_loop(0, lax.div(nb + 1, 2), outer_body, 0)
        plsc.subcore_barrier()
        pltpu.sync_copy(acc.at[pl.ds(s * TPC, TPC)],
                        nd_hbm.at[c, pl.ds(lo + s * TPC, TPC)])


def _tc2_body(nd_ref, bns_ref, bnb_ref, p_ref, pb_ref, pk_ref, mx2_ref):
    i = pl.program_id(0)
    nd = nd_ref[...]
    parts = []
    for h in range(4):
        half = h // 2
        num = nd[half, :, 32 * (h % 2):32 * (h % 2) + 32]
        den = nd[half, :, XC + (h % 2):XC + (h % 2) + 1]
        parts.append(jnp.maximum(num / (den + 1e-16), 0.0))
    hf = jnp.concatenate(parts, axis=1)
    hf = hf * bns_ref[...] + bnb_ref[...]
    hf = jnp.where(hf > 0, hf, jnp.exp(hf) - 1.0)
    pk = jnp.dot(hf, p_ref[...], preferred_element_type=jnp.float32)
    pk = pk + pb_ref[...]
    pk_ref[...] = pk
    bm = jnp.max(pk, axis=0, keepdims=True)
    bmf = jnp.concatenate([bm, jnp.full((1, 112), -1e30, jnp.float32)], axis=1)
    bmb = jnp.broadcast_to(bmf, (8, 128))

    @pl.when(i == 0)
    def _():
        mx2_ref[...] = bmb

    @pl.when(i != 0)
    def _():
        mx2_ref[...] = jnp.maximum(mx2_ref[...], bmb)


_tc2 = pl.pallas_call(
    _tc2_body,
    grid=(NP // R2,),
    in_specs=[
        pl.BlockSpec((2, R2, ACC), lambda i: (0, i, 0)),
        pl.BlockSpec((1, C), lambda i: (0, 0)),
        pl.BlockSpec((1, C), lambda i: (0, 0)),
        pl.BlockSpec((C, 16), lambda i: (0, 0)),
        pl.BlockSpec((1, 16), lambda i: (0, 0)),
    ],
    out_specs=[
        pl.BlockSpec((R2, 16), lambda i: (i, 0)),
        pl.BlockSpec((8, 128), lambda i: (0, 0)),
    ],
    out_shape=[
        jax.ShapeDtypeStruct((NP, 16), jnp.float32),
        jax.ShapeDtypeStruct((8, 128), jnp.float32),
    ],
)


@functools.partial(
    pl.kernel,
    out_type=jax.ShapeDtypeStruct((NP, 16), jnp.float32),
    mesh=_mesh,
    compiler_params=_sc_params,
    scratch_types=[
        pltpu.VMEM((BC,), jnp.int32),
        pltpu.VMEM((BC,), jnp.int32),
        pltpu.VMEM((BC,), jnp.int32),
        pltpu.VMEM((BC, 16), jnp.float32),
        pltpu.VMEM((BC, 16), jnp.float32),
        pltpu.VMEM((BC, 16), jnp.float32),
        pltpu.VMEM((1, 128), jnp.float32),
        pltpu.VMEM_SHARED((NH, 16), jnp.float32),
        pltpu.SemaphoreType.DMA,
    ],
)
def _sc_c(pk_hbm, srcp_hbm, dstp_hbm, mx2_hbm, nd2_hbm,
          sidx, didx, dloc, bufs, bufd, rout, mxv, acc, sem):
    c = lax.axis_index("c")
    s = lax.axis_index("s")
    lo = c * NH
    base0 = s * SB
    pltpu.sync_copy(mx2_hbm.at[pl.ds(0, 1)], mxv)
    mv = mxv[0, pl.ds(0, 16)]
    mm = mv[3] + mv[4]
    big_m = jnp.where(mm > 0, mm, 0.2 * mm)
    i16 = lax.iota(jnp.int32, 16)
    zero16 = jnp.zeros((16,), jnp.float32)

    def zrow(r, _):
        rout[r, pl.ds(0, 16)] = zero16
        return 0

    lax.fori_loop(0, BC, zrow, 0)
    for z in range(TPH // BC):
        pltpu.sync_copy(rout, acc.at[pl.ds(s * TPH + z * BC, BC)])
    rem = TPH % BC
    pltpu.sync_copy(rout.at[pl.ds(0, rem)],
                    acc.at[pl.ds(s * TPH + TPH - rem, rem)])
    plsc.subcore_barrier()

    def batch_body(bi, _):
        base = base0 + bi * BC
        pltpu.sync_copy(srcp_hbm.at[pl.ds(base, BC)], sidx)
        pltpu.sync_copy(dstp_hbm.at[pl.ds(base, BC)], didx)
        pltpu.async_copy(pk_hbm.at[sidx], bufs, sem).wait()
        pltpu.async_copy(pk_hbm.at[didx], bufd, sem).wait()

        def group_body(g, _):
            rows = g * 16 + i16
            sl = pl.ds(g * 16, 16)
            a2s = plsc.load_gather(bufs, [rows, jnp.full((16,), 3, jnp.int32)])
            a2d = plsc.load_gather(bufd, [rows, jnp.full((16,), 4, jnp.int32)])
            al = a2s + a2d
            al = jnp.where(al > 0, al, 0.2 * al)
            ev = jnp.exp(al - big_m)
            dv = didx[sl]
            eid = base + rows
            valid = (dv >= lo) & (dv < lo + NH) & (eid < E)
            ev = jnp.where(valid, ev, 0.0)
            dloc[sl] = jnp.where(valid, dv - lo, 0)
            for j in range(3):
                cj = jnp.full((16,), j, jnp.int32)
                xj = plsc.load_gather(bufs, [rows, cj])
                plsc.store_scatter(rout, [rows, cj], xj * ev)
            plsc.store_scatter(rout, [rows, jnp.full((16,), 3, jnp.int32)], ev)
            return 0

        lax.fori_loop(0, BC // 16, group_body, 0)
        pltpu.sync_copy(rout, acc.at[dloc], add=True)
        return 0

    lax.fori_loop(0, SB // BC, batch_body, 0)
    plsc.subcore_barrier()
    pltpu.sync_copy(acc.at[pl.ds(s * TPH, TPH)],
                    nd2_hbm.at[pl.ds(lo + s * TPH, TPH)])


def _tc3_body(nd2_ref, out_ref):
    nd = nd2_ref[...]
    den = nd[:, 3:4]
    o = [jnp.maximum(nd[:, j:j + 1] / (den + 1e-16), 0.0) for j in range(3)]
    om = jnp.maximum(jnp.maximum(o[0], o[1]), o[2])
    es = jnp.exp(o[0] - om) + jnp.exp(o[1] - om) + jnp.exp(o[2] - om)
    lse = jnp.log(es) + om
    cols = [o[j] - lse for j in range(3)]
    cols.append(jnp.zeros((R2, 13), jnp.float32))
    out_ref[...] = jnp.concatenate(cols, axis=1)


_tc3 = pl.pallas_call(
    _tc3_body,
    grid=(NP // R2,),
    in_specs=[pl.BlockSpec((R2, 16), lambda i: (i, 0))],
    out_specs=pl.BlockSpec((R2, 16), lambda i: (i, 0)),
    out_shape=jax.ShapeDtypeStruct((NP, 16), jnp.float32),
)


_SELS = np.zeros((C, 16), np.float32)
_SELD = np.zeros((C, 16), np.float32)
for _h in range(H):
    for _d in range(D):
        _SELS[_h * D + _d, _h] = 1.0
        _SELD[_h * D + _d, 4 + _h] = 1.0


def kernel(x_patient, x_symptom, x_dosha, ei_has_trait, ei_belongs_to,
           ei_similar_to, p1_proj_patient_w, p1_proj_patient_b,
           p1_proj_symptom_w, p1_proj_symptom_b, p1_proj_dosha_w,
           p1_proj_dosha_b, p1_src_has_trait, p1_dst_has_trait,
           p1_src_belongs_to, p1_dst_belongs_to, p1_src_similar_to,
           p1_dst_similar_to, p1_k_w, p1_k_b, p1_q, p2_proj_patient_w,
           p2_proj_patient_b, p2_proj_symptom_w, p2_proj_symptom_b,
           p2_proj_dosha_w, p2_proj_dosha_b, p2_src_has_trait,
           p2_dst_has_trait, p2_src_belongs_to, p2_dst_belongs_to,
           p2_src_similar_to, p2_dst_similar_to, p2_k_w, p2_k_b, p2_q,
           bn_w, bn_b, bn_rm, bn_rv):
    src = ei_similar_to[0]
    dst = ei_similar_to[1]
    pad = jnp.zeros((EP - E,), jnp.int32)
    srcp = jnp.concatenate([src, pad])
    dstp = jnp.concatenate([dst, pad])

    satt = p1_src_similar_to.reshape(1, C)
    datt = p1_dst_similar_to.reshape(1, C)
    xp, asd, mx = _tc1(x_patient, p1_proj_patient_w,
                       p1_proj_patient_b.reshape(1, C), satt, datt,
                       jnp.asarray(_SELS), jnp.asarray(_SELD))

    e_t = _sc_a(asd, srcp, dstp, mx)
    xp2d = xp.reshape(2 * N, XC)
    nd = _sc_b(xp2d, dstp, e_t)

    bns = (bn_w / jnp.sqrt(bn_rv + 1e-5)).reshape(1, C)
    bnb = (bn_b - bn_rm * bns[0]).reshape(1, C)
    s2 = p2_src_similar_to.reshape(3)
    d2 = p2_dst_similar_to.reshape(3)
    pmat = jnp.zeros((C, 16), jnp.float32)
    pmat = pmat.at[:, 0:3].set(p2_proj_patient_w)
    pmat = pmat.at[:, 3].set(p2_proj_patient_w @ s2)
    pmat = pmat.at[:, 4].set(p2_proj_patient_w @ d2)
    pb = jnp.zeros((16,), jnp.float32)
    pb = pb.at[0:3].set(p2_proj_patient_b)
    pb = pb.at[3].set(p2_proj_patient_b @ s2)
    pb = pb.at[4].set(p2_proj_patient_b @ d2)

    pk, mx2 = _tc2(nd, bns, bnb, pmat, pb.reshape(1, 16))
    nd2 = _sc_c(pk, srcp, dstp, mx2)
    out = _tc3(nd2)
    return out[:N, :3]


# speedup vs baseline: 1.1518x; 1.1518x over previous
import functools

import jax
import jax.numpy as jnp
import numpy as np
from jax import lax
from jax.experimental import pallas as pl
from jax.experimental.pallas import tpu as pltpu
from jax.experimental.pallas import tpu_sc as plsc

N = 50000
E = 150000
C = 128
H = 4
D = 32

NP = 50688
CHUNK = 12672
TPC = 792
NH = 25344
TPH = 1584
EP = 155648
ACC = 72
XC = 64

SA = EP // 32
BA = 128
SB = EP // 16
BC = 128
BB = 128
R1 = 2000
R2 = 4224

_mesh = plsc.VectorSubcoreMesh(core_axis_name="c", subcore_axis_name="s")
_sc_params = pltpu.CompilerParams(needs_layout_passes=False,
                                  use_tc_tiling_on_sc=False)


def _tc1_body(x_ref, w_ref, b_ref, sat_ref, dat_ref, sels_ref, seld_ref,
              xp_ref, asd_ref, mx_ref):
    i = pl.program_id(0)
    xp = jnp.dot(x_ref[...], w_ref[...], preferred_element_type=jnp.float32)
    xp = xp + b_ref[...]
    xp_ref[...] = xp
    asd = jnp.dot(xp * sat_ref[...], sels_ref[...],
                  preferred_element_type=jnp.float32)
    asd = asd + jnp.dot(xp * dat_ref[...], seld_ref[...],
                        preferred_element_type=jnp.float32)
    asd_ref[...] = asd
    bm = jnp.max(asd, axis=0, keepdims=True)
    bmf = jnp.concatenate([bm, jnp.full((1, 112), -1e30, jnp.float32)], axis=1)
    bmb = jnp.broadcast_to(bmf, (8, 128))

    @pl.when(i == 0)
    def _():
        mx_ref[...] = bmb

    @pl.when(i != 0)
    def _():
        mx_ref[...] = jnp.maximum(mx_ref[...], bmb)


_tc1 = pl.pallas_call(
    _tc1_body,
    grid=(N // R1,),
    in_specs=[
        pl.BlockSpec((R1, C), lambda i: (i, 0)),
        pl.BlockSpec((C, C), lambda i: (0, 0)),
        pl.BlockSpec((1, C), lambda i: (0, 0)),
        pl.BlockSpec((1, C), lambda i: (0, 0)),
        pl.BlockSpec((1, C), lambda i: (0, 0)),
        pl.BlockSpec((C, 16), lambda i: (0, 0)),
        pl.BlockSpec((C, 16), lambda i: (0, 0)),
    ],
    out_specs=[
        pl.BlockSpec((R1, C), lambda i: (i, 0)),
        pl.BlockSpec((R1, 16), lambda i: (i, 0)),
        pl.BlockSpec((8, 128), lambda i: (0, 0)),
    ],
    out_shape=[
        jax.ShapeDtypeStruct((N, C), jnp.float32),
        jax.ShapeDtypeStruct((N, 16), jnp.float32),
        jax.ShapeDtypeStruct((8, 128), jnp.float32),
    ],
)


@functools.partial(
    pl.kernel,
    out_type=jax.ShapeDtypeStruct((EP, 8), jnp.int32),
    mesh=_mesh,
    compiler_params=_sc_params,
    scratch_types=[
        pltpu.VMEM((BA,), jnp.int32),
        pltpu.VMEM((BA,), jnp.int32),
        pltpu.VMEM((BA, 16), jnp.float32),
        pltpu.VMEM((BA, 16), jnp.float32),
        pltpu.VMEM((BA, 8), jnp.int32),
        pltpu.VMEM((1, 128), jnp.float32),
        pltpu.SemaphoreType.DMA,
    ],
)
def _sc_a(asd_hbm, srcp_hbm, dstp_hbm, mx_hbm, e_hbm,
          sidx, didx, bufs, bufd, ebuf, mxv, sem):
    w = lax.axis_index("c") * 16 + lax.axis_index("s")
    base0 = w * SA
    pltpu.sync_copy(mx_hbm.at[pl.ds(0, 1)], mxv)
    mv = mxv[0, pl.ds(0, 16)]
    mm = [mv[h] + mv[4 + h] for h in range(4)]
    big_m = [jnp.where(m > 0, m, 0.2 * m) for m in mm]
    i16 = lax.iota(jnp.int32, 16)
    zi16 = jnp.zeros((16,), jnp.int32)

    def zpad(g, _):
        rows = g * 16 + i16
        plsc.store_scatter(ebuf, [rows, jnp.full((16,), 6, jnp.int32)], zi16)
        plsc.store_scatter(ebuf, [rows, jnp.full((16,), 7, jnp.int32)], zi16)
        return 0

    lax.fori_loop(0, BA // 16, zpad, 0)

    def batch_body(bi, _):
        base = base0 + bi * BA
        pltpu.sync_copy(srcp_hbm.at[pl.ds(base, BA)], sidx)
        pltpu.sync_copy(dstp_hbm.at[pl.ds(base, BA)], didx)
        pltpu.async_copy(asd_hbm.at[sidx], bufs, sem).wait()
        pltpu.async_copy(asd_hbm.at[didx], bufd, sem).wait()

        def group_body(g, _):
            rows = g * 16 + i16
            sl = pl.ds(g * 16, 16)
            eid = base + rows
            live = eid < E
            plsc.store_scatter(ebuf, [rows, jnp.full((16,), 0, jnp.int32)],
                               sidx[sl])
            plsc.store_scatter(ebuf, [rows, jnp.full((16,), 1, jnp.int32)],
                               didx[sl])
            for h in range(4):
                ch_s = jnp.full((16,), h, jnp.int32)
                ch_d = jnp.full((16,), 4 + h, jnp.int32)
                s = plsc.load_gather(bufs, [rows, ch_s])
                dd = plsc.load_gather(bufd, [rows, ch_d])
                al = s + dd
                al = jnp.where(al > 0, al, 0.2 * al)
                ev = jnp.exp(al - big_m[h])
                ev = jnp.where(live, ev, 0.0)
                plsc.store_scatter(ebuf, [rows, jnp.full((16,), 2 + h,
                                                         jnp.int32)],
                                   plsc.bitcast(ev, jnp.int32))
            return 0

        lax.fori_loop(0, BA // 16, group_body, 0)
        pltpu.sync_copy(ebuf, e_hbm.at[pl.ds(base, BA)])
        return 0

    lax.fori_loop(0, SA // BA, batch_body, 0)


@functools.partial(
    pl.kernel,
    out_type=jax.ShapeDtypeStruct((2, NP, ACC), jnp.float32),
    mesh=_mesh,
    compiler_params=_sc_params,
    scratch_types=[
        pltpu.VMEM((2, BB), jnp.int32),
        pltpu.VMEM((2, BB), jnp.int32),
        pltpu.VMEM((2, 2, BB), jnp.float32),
        pltpu.VMEM((2, BB, 8), jnp.int32),
        pltpu.VMEM((2, BB, XC), jnp.float32),
        pltpu.VMEM((2, BB, ACC), jnp.float32),
        pltpu.VMEM((SB + 16,), jnp.int32),
        pltpu.VMEM((SB + 16,), jnp.int32),
        pltpu.VMEM_SHARED((CHUNK, ACC), jnp.float32),
        pltpu.SemaphoreType.DMA,
        pltpu.SemaphoreType.DMA,
        pltpu.SemaphoreType.DMA,
        pltpu.SemaphoreType.DMA,
        pltpu.SemaphoreType.DMA,
        pltpu.SemaphoreType.DMA,
    ],
)
def _sc_b(xp2d_hbm, dstp_hbm, e_hbm, nd_hbm,
          sidx2, dloc2, ebuf2, ebat2, rin2, rout2, ldst, elist, acc,
          se0, se1, sr0, sr1, ss0, ss1):
    c = lax.axis_index("c")
    s = lax.axis_index("s")
    base0 = s * SB
    zero16 = jnp.zeros((16,), jnp.float32)
    sent16 = jnp.full((16,), EP - 1, jnp.int32)
    i16 = lax.iota(jnp.int32, 16)
    nchunks = NP // CHUNK
    slots = [(sidx2.at[0], dloc2.at[0], ebuf2.at[0], ebat2.at[0],
              rin2.at[0], rout2.at[0], se0, sr0, ss0),
             (sidx2.at[1], dloc2.at[1], ebuf2.at[1], ebat2.at[1],
              rin2.at[1], rout2.at[1], se1, sr1, ss1)]

    pltpu.sync_copy(dstp_hbm.at[pl.ds(base0, SB)], ldst.at[pl.ds(0, SB)])

    for pi in range(nchunks):
        lo = pi * CHUNK

        def prefill(g, _):
            elist[pl.ds(g * 16, 16)] = sent16
            return 0

        lax.fori_loop(0, SB // 16, prefill, 0)

        def bin_body(g, cnt):
            eid16 = base0 + g * 16 + i16
            dv = ldst[pl.ds(g * 16, 16)]
            mask = (dv >= lo) & (dv < lo + CHUNK)
            plsc.store_compressed(elist.at[pl.ds(cnt, 16)], eid16, mask=mask)
            pc = plsc.all_reduce_population_count(mask)
            return cnt + pc[0]

        cnt = lax.fori_loop(0, SB // 16, bin_body, 0)

        rout0 = rout2.at[0]

        def zrow(r, _):
            for k in range(4):
                rout0[r, pl.ds(16 * k, 16)] = zero16
            rout0[r, pl.ds(ACC - 16, 16)] = zero16
            return 0

        lax.fori_loop(0, BB, zrow, 0)
        for z in range(TPC // BB):
            pltpu.sync_copy(rout0, acc.at[pl.ds(s * TPC + z * BB, BB)])
        rem = TPC % BB
        pltpu.sync_copy(rout0.at[pl.ds(0, rem)],
                        acc.at[pl.ds(s * TPC + TPC - rem, rem)])
        plsc.subcore_barrier()

        nb = lax.div(cnt + (BB - 1), BB)

        def ebat_copy(bi, slot):
            return pltpu.make_async_copy(
                e_hbm.at[elist.at[pl.ds(bi * BB, BB)]], ebat2.at[slot],
                [se0, se1][slot])

        def build_group(bi, slot):
            sidx, dloc, ebuf, ebat = (sidx2.at[slot], dloc2.at[slot],
                                      ebuf2.at[slot], ebat2.at[slot])

            def group_body(g, _):
                sl = pl.ds(g * 16, 16)
                rows = g * 16 + i16
                sv = plsc.load_gather(ebat,
                                      [rows, jnp.full((16,), 0, jnp.int32)])
                dv = plsc.load_gather(ebat,
                                      [rows, jnp.full((16,), 1, jnp.int32)])
                sidx[sl] = sv * 2 + c
                dloc[sl] = jnp.maximum(dv - lo, 0)
                for h in range(2):
                    evi = plsc.load_gather(
                        ebat, [rows, jnp.full((16,), 2 + 2 * c + h,
                                              jnp.int32)])
                    ebuf[h, sl] = plsc.bitcast(evi, jnp.float32)
                return 0

            lax.fori_loop(0, BB // 16, group_body, 0)
            pltpu.async_copy(xp2d_hbm.at[sidx2.at[slot]], rin2.at[slot],
                             [sr0, sr1][slot])

        @pl.when(nb > 0)
        def _():
            pltpu.async_copy(e_hbm.at[elist.at[pl.ds(0, BB)]], ebat2.at[0],
                             se0)
            ebat_copy(0, 0).wait()
            build_group(0, 0)

        @pl.when(nb > 1)
        def _():
            pltpu.async_copy(e_hbm.at[elist.at[pl.ds(BB, BB)]], ebat2.at[1],
                             se1)

        def outer_body(bo, _):
            for u in range(2):
                sidx, dloc, ebuf, ebat, rin, rout, sem_e, sem_r, sem_s = \
                    slots[u]
                _, dloc_o, _, _, _, rout_o, _, _, sem_so = slots[1 - u]
                bi = 2 * bo + u

                @pl.when(bi < nb)
                def _():
                    pltpu.make_async_copy(xp2d_hbm.at[sidx], rin,
                                          sem_r).wait()

                    @pl.when(bi + 2 < nb)
                    def _():
                        pltpu.async_copy(
                            e_hbm.at[elist.at[pl.ds((bi + 2) * BB, BB)]],
                            ebat, sem_e)

                    @pl.when(bi >= 1)
                    def _():
                        pltpu.make_async_copy(rout_o, acc.at[dloc_o],
                                              sem_so).wait()

                    @pl.when(bi + 1 < nb)
                    def _():
                        ebat_copy(bi + 1, 1 - u).wait()
                        build_group(bi + 1, 1 - u)

                    def scale_body(g, _):
                        sl = pl.ds(g * 16, 16)
                        rows = g * 16 + i16
                        evs = []
                        for h in range(2):
                            ev = ebuf[h, sl]
                            evs.append(ev)
                            plsc.store_scatter(
                                rout, [rows, jnp.full((16,), XC + h,
                                                      jnp.int32)], ev)
                        for col in range(XC):
                            cc = jnp.full((16,), col, jnp.int32)
                            vals = plsc.load_gather(rin, [rows, cc])
                            plsc.store_scatter(rout, [rows, cc],
                                               vals * evs[col // 32])
                        return 0

                    lax.fori_loop(0, BB // 16, scale_body, 0)
                    pltpu.async_copy(rout, acc.at[dloc], sem_s, add=True)
            return 0

        lax.fori_loop(0, lax.div(nb + 1, 2), outer_body, 0)
        for u in range(2):
            sidx, dloc, ebuf, ebat, rin, rout, sem_e, sem_r, sem_s = slots[u]

            @pl.when((nb > 0) & (lax.rem(nb + 1, 2) == u))
            def _():
                pltpu.make_async_copy(rout, acc.at[dloc], sem_s).wait()

        plsc.subcore_barrier()
        pltpu.sync_copy(acc.at[pl.ds(s * TPC, TPC)],
                        nd_hbm.at[c, pl.ds(lo + s * TPC, TPC)])


def _tc2_body(nd_ref, bns_ref, bnb_ref, p_ref, pb_ref, pk_ref, mx2_ref):
    i = pl.program_id(0)
    nd = nd_ref[...]
    parts = []
    for h in range(4):
        half = h // 2
        num = nd[half, :, 32 * (h % 2):32 * (h % 2) + 32]
        den = nd[half, :, XC + (h % 2):XC + (h % 2) + 1]
        parts.append(jnp.maximum(num / (den + 1e-16), 0.0))
    hf = jnp.concatenate(parts, axis=1)
    hf = hf * bns_ref[...] + bnb_ref[...]
    hf = jnp.where(hf > 0, hf, jnp.exp(hf) - 1.0)
    pk = jnp.dot(hf, p_ref[...], preferred_element_type=jnp.float32)
    pk = pk + pb_ref[...]
    pk_ref[...] = pk
    bm = jnp.max(pk, axis=0, keepdims=True)
    bmf = jnp.concatenate([bm, jnp.full((1, 112), -1e30, jnp.float32)], axis=1)
    bmb = jnp.broadcast_to(bmf, (8, 128))

    @pl.when(i == 0)
    def _():
        mx2_ref[...] = bmb

    @pl.when(i != 0)
    def _():
        mx2_ref[...] = jnp.maximum(mx2_ref[...], bmb)


_tc2 = pl.pallas_call(
    _tc2_body,
    grid=(NP // R2,),
    in_specs=[
        pl.BlockSpec((2, R2, ACC), lambda i: (0, i, 0)),
        pl.BlockSpec((1, C), lambda i: (0, 0)),
        pl.BlockSpec((1, C), lambda i: (0, 0)),
        pl.BlockSpec((C, 16), lambda i: (0, 0)),
        pl.BlockSpec((1, 16), lambda i: (0, 0)),
    ],
    out_specs=[
        pl.BlockSpec((R2, 16), lambda i: (i, 0)),
        pl.BlockSpec((8, 128), lambda i: (0, 0)),
    ],
    out_shape=[
        jax.ShapeDtypeStruct((NP, 16), jnp.float32),
        jax.ShapeDtypeStruct((8, 128), jnp.float32),
    ],
)


@functools.partial(
    pl.kernel,
    out_type=jax.ShapeDtypeStruct((NP, 16), jnp.float32),
    mesh=_mesh,
    compiler_params=_sc_params,
    scratch_types=[
        pltpu.VMEM((BC,), jnp.int32),
        pltpu.VMEM((BC,), jnp.int32),
        pltpu.VMEM((BC,), jnp.int32),
        pltpu.VMEM((BC, 16), jnp.float32),
        pltpu.VMEM((BC, 16), jnp.float32),
        pltpu.VMEM((BC, 16), jnp.float32),
        pltpu.VMEM((1, 128), jnp.float32),
        pltpu.VMEM_SHARED((NH, 16), jnp.float32),
        pltpu.SemaphoreType.DMA,
    ],
)
def _sc_c(pk_hbm, srcp_hbm, dstp_hbm, mx2_hbm, nd2_hbm,
          sidx, didx, dloc, bufs, bufd, rout, mxv, acc, sem):
    c = lax.axis_index("c")
    s = lax.axis_index("s")
    lo = c * NH
    base0 = s * SB
    pltpu.sync_copy(mx2_hbm.at[pl.ds(0, 1)], mxv)
    mv = mxv[0, pl.ds(0, 16)]
    mm = mv[3] + mv[4]
    big_m = jnp.where(mm > 0, mm, 0.2 * mm)
    i16 = lax.iota(jnp.int32, 16)
    zero16 = jnp.zeros((16,), jnp.float32)

    def zrow(r, _):
        rout[r, pl.ds(0, 16)] = zero16
        return 0

    lax.fori_loop(0, BC, zrow, 0)
    for z in range(TPH // BC):
        pltpu.sync_copy(rout, acc.at[pl.ds(s * TPH + z * BC, BC)])
    rem = TPH % BC
    pltpu.sync_copy(rout.at[pl.ds(0, rem)],
                    acc.at[pl.ds(s * TPH + TPH - rem, rem)])
    plsc.subcore_barrier()

    def batch_body(bi, _):
        base = base0 + bi * BC
        pltpu.sync_copy(srcp_hbm.at[pl.ds(base, BC)], sidx)
        pltpu.sync_copy(dstp_hbm.at[pl.ds(base, BC)], didx)
        pltpu.async_copy(pk_hbm.at[sidx], bufs, sem).wait()
        pltpu.async_copy(pk_hbm.at[didx], bufd, sem).wait()

        def group_body(g, _):
            rows = g * 16 + i16
            sl = pl.ds(g * 16, 16)
            a2s = plsc.load_gather(bufs, [rows, jnp.full((16,), 3, jnp.int32)])
            a2d = plsc.load_gather(bufd, [rows, jnp.full((16,), 4, jnp.int32)])
            al = a2s + a2d
            al = jnp.where(al > 0, al, 0.2 * al)
            ev = jnp.exp(al - big_m)
            dv = didx[sl]
            eid = base + rows
            valid = (dv >= lo) & (dv < lo + NH) & (eid < E)
            ev = jnp.where(valid, ev, 0.0)
            dloc[sl] = jnp.where(valid, dv - lo, 0)
            for j in range(3):
                cj = jnp.full((16,), j, jnp.int32)
                xj = plsc.load_gather(bufs, [rows, cj])
                plsc.store_scatter(rout, [rows, cj], xj * ev)
            plsc.store_scatter(rout, [rows, jnp.full((16,), 3, jnp.int32)], ev)
            return 0

        lax.fori_loop(0, BC // 16, group_body, 0)
        pltpu.sync_copy(rout, acc.at[dloc], add=True)
        return 0

    lax.fori_loop(0, SB // BC, batch_body, 0)
    plsc.subcore_barrier()
    pltpu.sync_copy(acc.at[pl.ds(s * TPH, TPH)],
                    nd2_hbm.at[pl.ds(lo + s * TPH, TPH)])


def _tc3_body(nd2_ref, out_ref):
    nd = nd2_ref[...]
    den = nd[:, 3:4]
    o = [jnp.maximum(nd[:, j:j + 1] / (den + 1e-16), 0.0) for j in range(3)]
    om = jnp.maximum(jnp.maximum(o[0], o[1]), o[2])
    es = jnp.exp(o[0] - om) + jnp.exp(o[1] - om) + jnp.exp(o[2] - om)
    lse = jnp.log(es) + om
    cols = [o[j] - lse for j in range(3)]
    cols.append(jnp.zeros((R2, 13), jnp.float32))
    out_ref[...] = jnp.concatenate(cols, axis=1)


_tc3 = pl.pallas_call(
    _tc3_body,
    grid=(NP // R2,),
    in_specs=[pl.BlockSpec((R2, 16), lambda i: (i, 0))],
    out_specs=pl.BlockSpec((R2, 16), lambda i: (i, 0)),
    out_shape=jax.ShapeDtypeStruct((NP, 16), jnp.float32),
)


_SELS = np.zeros((C, 16), np.float32)
_SELD = np.zeros((C, 16), np.float32)
for _h in range(H):
    for _d in range(D):
        _SELS[_h * D + _d, _h] = 1.0
        _SELD[_h * D + _d, 4 + _h] = 1.0


def kernel(x_patient, x_symptom, x_dosha, ei_has_trait, ei_belongs_to,
           ei_similar_to, p1_proj_patient_w, p1_proj_patient_b,
           p1_proj_symptom_w, p1_proj_symptom_b, p1_proj_dosha_w,
           p1_proj_dosha_b, p1_src_has_trait, p1_dst_has_trait,
           p1_src_belongs_to, p1_dst_belongs_to, p1_src_similar_to,
           p1_dst_similar_to, p1_k_w, p1_k_b, p1_q, p2_proj_patient_w,
           p2_proj_patient_b, p2_proj_symptom_w, p2_proj_symptom_b,
           p2_proj_dosha_w, p2_proj_dosha_b, p2_src_has_trait,
           p2_dst_has_trait, p2_src_belongs_to, p2_dst_belongs_to,
           p2_src_similar_to, p2_dst_similar_to, p2_k_w, p2_k_b, p2_q,
           bn_w, bn_b, bn_rm, bn_rv):
    src = ei_similar_to[0]
    dst = ei_similar_to[1]
    pad = jnp.zeros((EP - E,), jnp.int32)
    srcp = jnp.concatenate([src, pad])
    dstp = jnp.concatenate([dst, pad])

    satt = p1_src_similar_to.reshape(1, C)
    datt = p1_dst_similar_to.reshape(1, C)
    xp, asd, mx = _tc1(x_patient, p1_proj_patient_w,
                       p1_proj_patient_b.reshape(1, C), satt, datt,
                       jnp.asarray(_SELS), jnp.asarray(_SELD))

    e_t = _sc_a(asd, srcp, dstp, mx)
    xp2d = xp.reshape(2 * N, XC)
    nd = _sc_b(xp2d, dstp, e_t)

    bns = (bn_w / jnp.sqrt(bn_rv + 1e-5)).reshape(1, C)
    bnb = (bn_b - bn_rm * bns[0]).reshape(1, C)
    s2 = p2_src_similar_to.reshape(3)
    d2 = p2_dst_similar_to.reshape(3)
    pmat = jnp.zeros((C, 16), jnp.float32)
    pmat = pmat.at[:, 0:3].set(p2_proj_patient_w)
    pmat = pmat.at[:, 3].set(p2_proj_patient_w @ s2)
    pmat = pmat.at[:, 4].set(p2_proj_patient_w @ d2)
    pb = jnp.zeros((16,), jnp.float32)
    pb = pb.at[0:3].set(p2_proj_patient_b)
    pb = pb.at[3].set(p2_proj_patient_b @ s2)
    pb = pb.at[4].set(p2_proj_patient_b @ d2)

    pk, mx2 = _tc2(nd, bns, bnb, pmat, pb.reshape(1, 16))
    nd2 = _sc_c(pk, srcp, dstp, mx2)
    out = _tc3(nd2)
    return out[:N, :3]


# speedup vs baseline: 1.2008x; 1.0425x over previous
import functools

import jax
import jax.numpy as jnp
import numpy as np
from jax import lax
from jax.experimental import pallas as pl
from jax.experimental.pallas import tpu as pltpu
from jax.experimental.pallas import tpu_sc as plsc

N = 50000
E = 150000
C = 128
H = 4
D = 32

NP = 50688
CHUNK = 12672
TPC = 792
NH = 25344
TPH = 1584
EP = 155648
ACC = 72
XC = 64

SA = EP // 32
BA = 128
SB = EP // 16
BC = 128
BB = 128
R1 = 2000
R2 = 4224

_mesh = plsc.VectorSubcoreMesh(core_axis_name="c", subcore_axis_name="s")
_sc_params = pltpu.CompilerParams(needs_layout_passes=False,
                                  use_tc_tiling_on_sc=False)


def _tc1_body(x_ref, w_ref, b_ref, sat_ref, dat_ref, sels_ref, seld_ref,
              xp_ref, asd_ref, mx_ref):
    i = pl.program_id(0)
    xp = jnp.dot(x_ref[...], w_ref[...], preferred_element_type=jnp.float32)
    xp = xp + b_ref[...]
    xp_ref[...] = xp
    asd = jnp.dot(xp * sat_ref[...], sels_ref[...],
                  preferred_element_type=jnp.float32)
    asd = asd + jnp.dot(xp * dat_ref[...], seld_ref[...],
                        preferred_element_type=jnp.float32)
    asd_ref[...] = asd
    bm = jnp.max(asd, axis=0, keepdims=True)
    bmf = jnp.concatenate([bm, jnp.full((1, 112), -1e30, jnp.float32)], axis=1)
    bmb = jnp.broadcast_to(bmf, (8, 128))

    @pl.when(i == 0)
    def _():
        mx_ref[...] = bmb

    @pl.when(i != 0)
    def _():
        mx_ref[...] = jnp.maximum(mx_ref[...], bmb)


_tc1 = pl.pallas_call(
    _tc1_body,
    grid=(N // R1,),
    in_specs=[
        pl.BlockSpec((R1, C), lambda i: (i, 0)),
        pl.BlockSpec((C, C), lambda i: (0, 0)),
        pl.BlockSpec((1, C), lambda i: (0, 0)),
        pl.BlockSpec((1, C), lambda i: (0, 0)),
        pl.BlockSpec((1, C), lambda i: (0, 0)),
        pl.BlockSpec((C, 16), lambda i: (0, 0)),
        pl.BlockSpec((C, 16), lambda i: (0, 0)),
    ],
    out_specs=[
        pl.BlockSpec((R1, C), lambda i: (i, 0)),
        pl.BlockSpec((R1, 16), lambda i: (i, 0)),
        pl.BlockSpec((8, 128), lambda i: (0, 0)),
    ],
    out_shape=[
        jax.ShapeDtypeStruct((N, C), jnp.float32),
        jax.ShapeDtypeStruct((N, 16), jnp.float32),
        jax.ShapeDtypeStruct((8, 128), jnp.float32),
    ],
)


@functools.partial(
    pl.kernel,
    out_type=jax.ShapeDtypeStruct((EP, 8), jnp.int32),
    mesh=_mesh,
    compiler_params=_sc_params,
    scratch_types=[
        pltpu.VMEM((BA,), jnp.int32),
        pltpu.VMEM((BA,), jnp.int32),
        pltpu.VMEM((BA, 16), jnp.float32),
        pltpu.VMEM((BA, 16), jnp.float32),
        pltpu.VMEM((BA, 8), jnp.int32),
        pltpu.VMEM((1, 128), jnp.float32),
        pltpu.SemaphoreType.DMA,
        pltpu.SemaphoreType.DMA,
    ],
)
def _sc_a(asd_hbm, srcp_hbm, dstp_hbm, mx_hbm, e_hbm,
          sidx, didx, bufs, bufd, ebuf, mxv, sem, sem2):
    w = lax.axis_index("c") * 16 + lax.axis_index("s")
    base0 = w * SA
    pltpu.sync_copy(mx_hbm.at[pl.ds(0, 1)], mxv)
    mv = mxv[0, pl.ds(0, 16)]
    mm = [mv[h] + mv[4 + h] for h in range(4)]
    big_m = [jnp.where(m > 0, m, 0.2 * m) for m in mm]
    i16 = lax.iota(jnp.int32, 16)
    zi16 = jnp.zeros((16,), jnp.int32)

    def zpad(g, _):
        rows = g * 16 + i16
        plsc.store_scatter(ebuf, [rows, jnp.full((16,), 6, jnp.int32)], zi16)
        plsc.store_scatter(ebuf, [rows, jnp.full((16,), 7, jnp.int32)], zi16)
        return 0

    lax.fori_loop(0, BA // 16, zpad, 0)

    def batch_body(bi, _):
        base = base0 + bi * BA
        pltpu.sync_copy(srcp_hbm.at[pl.ds(base, BA)], sidx)
        pltpu.sync_copy(dstp_hbm.at[pl.ds(base, BA)], didx)
        pltpu.async_copy(asd_hbm.at[sidx], bufs, sem)
        pltpu.async_copy(asd_hbm.at[didx], bufd, sem2)
        pltpu.make_async_copy(asd_hbm.at[sidx], bufs, sem).wait()
        pltpu.make_async_copy(asd_hbm.at[didx], bufd, sem2).wait()

        def group_body(g, _):
            rows = g * 16 + i16
            sl = pl.ds(g * 16, 16)
            eid = base + rows
            live = eid < E
            plsc.store_scatter(ebuf, [rows, jnp.full((16,), 0, jnp.int32)],
                               sidx[sl])
            plsc.store_scatter(ebuf, [rows, jnp.full((16,), 1, jnp.int32)],
                               didx[sl])
            for h in range(4):
                ch_s = jnp.full((16,), h, jnp.int32)
                ch_d = jnp.full((16,), 4 + h, jnp.int32)
                s = plsc.load_gather(bufs, [rows, ch_s])
                dd = plsc.load_gather(bufd, [rows, ch_d])
                al = s + dd
                al = jnp.where(al > 0, al, 0.2 * al)
                ev = jnp.exp(al - big_m[h])
                ev = jnp.where(live, ev, 0.0)
                plsc.store_scatter(ebuf, [rows, jnp.full((16,), 2 + h,
                                                         jnp.int32)],
                                   plsc.bitcast(ev, jnp.int32))
            return 0

        lax.fori_loop(0, BA // 16, group_body, 0)
        pltpu.sync_copy(ebuf, e_hbm.at[pl.ds(base, BA)])
        return 0

    lax.fori_loop(0, SA // BA, batch_body, 0)


@functools.partial(
    pl.kernel,
    out_type=jax.ShapeDtypeStruct((2, NP, ACC), jnp.float32),
    mesh=_mesh,
    compiler_params=_sc_params,
    scratch_types=[
        pltpu.VMEM((2, BB), jnp.int32),
        pltpu.VMEM((2, BB), jnp.int32),
        pltpu.VMEM((2, 2, BB), jnp.float32),
        pltpu.VMEM((2, BB, 8), jnp.int32),
        pltpu.VMEM((2, BB, XC), jnp.float32),
        pltpu.VMEM((2, BB, ACC), jnp.float32),
        pltpu.VMEM((SB + 16,), jnp.int32),
        pltpu.VMEM((SB + 16,), jnp.int32),
        pltpu.VMEM_SHARED((CHUNK, ACC), jnp.float32),
        pltpu.SemaphoreType.DMA,
        pltpu.SemaphoreType.DMA,
        pltpu.SemaphoreType.DMA,
        pltpu.SemaphoreType.DMA,
        pltpu.SemaphoreType.DMA,
        pltpu.SemaphoreType.DMA,
    ],
)
def _sc_b(xp2d_hbm, dstp_hbm, e_hbm, nd_hbm,
          sidx2, dloc2, ebuf2, ebat2, rin2, rout2, ldst, elist, acc,
          se0, se1, sr0, sr1, ss0, ss1):
    c = lax.axis_index("c")
    s = lax.axis_index("s")
    base0 = s * SB
    zero16 = jnp.zeros((16,), jnp.float32)
    sent16 = jnp.full((16,), EP - 1, jnp.int32)
    i16 = lax.iota(jnp.int32, 16)
    nchunks = NP // CHUNK
    slots = [(sidx2.at[0], dloc2.at[0], ebuf2.at[0], ebat2.at[0],
              rin2.at[0], rout2.at[0], se0, sr0, ss0),
             (sidx2.at[1], dloc2.at[1], ebuf2.at[1], ebat2.at[1],
              rin2.at[1], rout2.at[1], se1, sr1, ss1)]

    pltpu.sync_copy(dstp_hbm.at[pl.ds(base0, SB)], ldst.at[pl.ds(0, SB)])

    for pi in range(nchunks):
        lo = pi * CHUNK

        def prefill(g, _):
            elist[pl.ds(g * 16, 16)] = sent16
            return 0

        lax.fori_loop(0, SB // 16, prefill, 0)

        def bin_body(g, cnt):
            eid16 = base0 + g * 16 + i16
            dv = ldst[pl.ds(g * 16, 16)]
            mask = (dv >= lo) & (dv < lo + CHUNK)
            plsc.store_compressed(elist.at[pl.ds(cnt, 16)], eid16, mask=mask)
            pc = plsc.all_reduce_population_count(mask)
            return cnt + pc[0]

        cnt = lax.fori_loop(0, SB // 16, bin_body, 0)

        rout0 = rout2.at[0]

        def zrow(r, _):
            for k in range(4):
                rout0[r, pl.ds(16 * k, 16)] = zero16
            rout0[r, pl.ds(ACC - 16, 16)] = zero16
            return 0

        lax.fori_loop(0, BB, zrow, 0)
        for z in range(TPC // BB):
            pltpu.sync_copy(rout0, acc.at[pl.ds(s * TPC + z * BB, BB)])
        rem = TPC % BB
        pltpu.sync_copy(rout0.at[pl.ds(0, rem)],
                        acc.at[pl.ds(s * TPC + TPC - rem, rem)])
        plsc.subcore_barrier()

        nb = lax.div(cnt + (BB - 1), BB)

        def ebat_copy(bi, slot):
            return pltpu.make_async_copy(
                e_hbm.at[elist.at[pl.ds(bi * BB, BB)]], ebat2.at[slot],
                [se0, se1][slot])

        def build_group(bi, slot):
            sidx, dloc, ebuf, ebat = (sidx2.at[slot], dloc2.at[slot],
                                      ebuf2.at[slot], ebat2.at[slot])

            def group_body(g, _):
                sl = pl.ds(g * 16, 16)
                rows = g * 16 + i16
                sv = plsc.load_gather(ebat,
                                      [rows, jnp.full((16,), 0, jnp.int32)])
                dv = plsc.load_gather(ebat,
                                      [rows, jnp.full((16,), 1, jnp.int32)])
                sidx[sl] = sv * 2 + c
                dloc[sl] = jnp.maximum(dv - lo, 0)
                for h in range(2):
                    evi = plsc.load_gather(
                        ebat, [rows, jnp.full((16,), 2 + 2 * c + h,
                                              jnp.int32)])
                    ebuf[h, sl] = plsc.bitcast(evi, jnp.float32)
                return 0

            lax.fori_loop(0, BB // 16, group_body, 0)
            pltpu.async_copy(xp2d_hbm.at[sidx2.at[slot]], rin2.at[slot],
                             [sr0, sr1][slot])

        @pl.when(nb > 0)
        def _():
            pltpu.async_copy(e_hbm.at[elist.at[pl.ds(0, BB)]], ebat2.at[0],
                             se0)
            ebat_copy(0, 0).wait()
            build_group(0, 0)

        @pl.when(nb > 1)
        def _():
            pltpu.async_copy(e_hbm.at[elist.at[pl.ds(BB, BB)]], ebat2.at[1],
                             se1)

        def outer_body(bo, _):
            for u in range(2):
                sidx, dloc, ebuf, ebat, rin, rout, sem_e, sem_r, sem_s = \
                    slots[u]
                _, dloc_o, _, _, _, rout_o, _, _, sem_so = slots[1 - u]
                bi = 2 * bo + u

                @pl.when(bi < nb)
                def _():
                    pltpu.make_async_copy(xp2d_hbm.at[sidx], rin,
                                          sem_r).wait()

                    @pl.when(bi + 2 < nb)
                    def _():
                        pltpu.async_copy(
                            e_hbm.at[elist.at[pl.ds((bi + 2) * BB, BB)]],
                            ebat, sem_e)

                    @pl.when(bi >= 1)
                    def _():
                        pltpu.make_async_copy(rout_o, acc.at[dloc_o],
                                              sem_so).wait()

                    @pl.when(bi + 1 < nb)
                    def _():
                        ebat_copy(bi + 1, 1 - u).wait()
                        build_group(bi + 1, 1 - u)

                    def scale_body(g, _):
                        sl = pl.ds(g * 16, 16)
                        rows = g * 16 + i16
                        evs = []
                        for h in range(2):
                            ev = ebuf[h, sl]
                            evs.append(ev)
                            plsc.store_scatter(
                                rout, [rows, jnp.full((16,), XC + h,
                                                      jnp.int32)], ev)
                        for col in range(XC):
                            cc = jnp.full((16,), col, jnp.int32)
                            vals = plsc.load_gather(rin, [rows, cc])
                            plsc.store_scatter(rout, [rows, cc],
                                               vals * evs[col // 32])
                        return 0

                    lax.fori_loop(0, BB // 16, scale_body, 0)
                    pltpu.async_copy(rout, acc.at[dloc], sem_s, add=True)
            return 0

        lax.fori_loop(0, lax.div(nb + 1, 2), outer_body, 0)
        for u in range(2):
            sidx, dloc, ebuf, ebat, rin, rout, sem_e, sem_r, sem_s = slots[u]

            @pl.when((nb > 0) & (lax.rem(nb + 1, 2) == u))
            def _():
                pltpu.make_async_copy(rout, acc.at[dloc], sem_s).wait()

        plsc.subcore_barrier()
        pltpu.sync_copy(acc.at[pl.ds(s * TPC, TPC)],
                        nd_hbm.at[c, pl.ds(lo + s * TPC, TPC)])


def _tc2_body(nd_ref, bns_ref, bnb_ref, p_ref, pb_ref, pk_ref, mx2_ref):
    i = pl.program_id(0)
    nd = nd_ref[...]
    parts = []
    for h in range(4):
        half = h // 2
        num = nd[half, :, 32 * (h % 2):32 * (h % 2) + 32]
        den = nd[half, :, XC + (h % 2):XC + (h % 2) + 1]
        parts.append(jnp.maximum(num / (den + 1e-16), 0.0))
    hf = jnp.concatenate(parts, axis=1)
    hf = hf * bns_ref[...] + bnb_ref[...]
    hf = jnp.where(hf > 0, hf, jnp.exp(hf) - 1.0)
    pk = jnp.dot(hf, p_ref[...], preferred_element_type=jnp.float32)
    pk = pk + pb_ref[...]
    pk_ref[...] = pk
    bm = jnp.max(pk, axis=0, keepdims=True)
    bmf = jnp.concatenate([bm, jnp.full((1, 112), -1e30, jnp.float32)], axis=1)
    bmb = jnp.broadcast_to(bmf, (8, 128))

    @pl.when(i == 0)
    def _():
        mx2_ref[...] = bmb

    @pl.when(i != 0)
    def _():
        mx2_ref[...] = jnp.maximum(mx2_ref[...], bmb)


_tc2 = pl.pallas_call(
    _tc2_body,
    grid=(NP // R2,),
    in_specs=[
        pl.BlockSpec((2, R2, ACC), lambda i: (0, i, 0)),
        pl.BlockSpec((1, C), lambda i: (0, 0)),
        pl.BlockSpec((1, C), lambda i: (0, 0)),
        pl.BlockSpec((C, 16), lambda i: (0, 0)),
        pl.BlockSpec((1, 16), lambda i: (0, 0)),
    ],
    out_specs=[
        pl.BlockSpec((R2, 16), lambda i: (i, 0)),
        pl.BlockSpec((8, 128), lambda i: (0, 0)),
    ],
    out_shape=[
        jax.ShapeDtypeStruct((NP, 16), jnp.float32),
        jax.ShapeDtypeStruct((8, 128), jnp.float32),
    ],
)


@functools.partial(
    pl.kernel,
    out_type=jax.ShapeDtypeStruct((NP, 16), jnp.float32),
    mesh=_mesh,
    compiler_params=_sc_params,
    scratch_types=[
        pltpu.VMEM((BC,), jnp.int32),
        pltpu.VMEM((BC,), jnp.int32),
        pltpu.VMEM((BC,), jnp.int32),
        pltpu.VMEM((BC, 16), jnp.float32),
        pltpu.VMEM((BC, 16), jnp.float32),
        pltpu.VMEM((BC, 16), jnp.float32),
        pltpu.VMEM((1, 128), jnp.float32),
        pltpu.VMEM_SHARED((NH, 16), jnp.float32),
        pltpu.SemaphoreType.DMA,
        pltpu.SemaphoreType.DMA,
        pltpu.SemaphoreType.DMA,
    ],
)
def _sc_c(pk_hbm, srcp_hbm, dstp_hbm, mx2_hbm, nd2_hbm,
          sidx, didx, dloc, bufs, bufd, rout, mxv, acc, sem, sem2, sem3):
    c = lax.axis_index("c")
    s = lax.axis_index("s")
    lo = c * NH
    base0 = s * SB
    pltpu.sync_copy(mx2_hbm.at[pl.ds(0, 1)], mxv)
    mv = mxv[0, pl.ds(0, 16)]
    mm = mv[3] + mv[4]
    big_m = jnp.where(mm > 0, mm, 0.2 * mm)
    i16 = lax.iota(jnp.int32, 16)
    zero16 = jnp.zeros((16,), jnp.float32)

    def zrow(r, _):
        rout[r, pl.ds(0, 16)] = zero16
        return 0

    lax.fori_loop(0, BC, zrow, 0)
    for z in range(TPH // BC):
        pltpu.sync_copy(rout, acc.at[pl.ds(s * TPH + z * BC, BC)])
    rem = TPH % BC
    pltpu.sync_copy(rout.at[pl.ds(0, rem)],
                    acc.at[pl.ds(s * TPH + TPH - rem, rem)])
    plsc.subcore_barrier()

    def batch_body(bi, _):
        base = base0 + bi * BC
        pltpu.sync_copy(srcp_hbm.at[pl.ds(base, BC)], sidx)
        pltpu.sync_copy(dstp_hbm.at[pl.ds(base, BC)], didx)
        pltpu.async_copy(pk_hbm.at[sidx], bufs, sem)
        pltpu.async_copy(pk_hbm.at[didx], bufd, sem2)

        @pl.when(bi >= 1)
        def _():
            pltpu.make_async_copy(rout, acc.at[dloc], sem3).wait()

        pltpu.make_async_copy(pk_hbm.at[sidx], bufs, sem).wait()
        pltpu.make_async_copy(pk_hbm.at[didx], bufd, sem2).wait()

        def group_body(g, _):
            rows = g * 16 + i16
            sl = pl.ds(g * 16, 16)
            a2s = plsc.load_gather(bufs, [rows, jnp.full((16,), 3, jnp.int32)])
            a2d = plsc.load_gather(bufd, [rows, jnp.full((16,), 4, jnp.int32)])
            al = a2s + a2d
            al = jnp.where(al > 0, al, 0.2 * al)
            ev = jnp.exp(al - big_m)
            dv = didx[sl]
            eid = base + rows
            valid = (dv >= lo) & (dv < lo + NH) & (eid < E)
            ev = jnp.where(valid, ev, 0.0)
            dloc[sl] = jnp.where(valid, dv - lo, 0)
            for j in range(3):
                cj = jnp.full((16,), j, jnp.int32)
                xj = plsc.load_gather(bufs, [rows, cj])
                plsc.store_scatter(rout, [rows, cj], xj * ev)
            plsc.store_scatter(rout, [rows, jnp.full((16,), 3, jnp.int32)], ev)
            return 0

        lax.fori_loop(0, BC // 16, group_body, 0)
        pltpu.async_copy(rout, acc.at[dloc], sem3, add=True)
        return 0

    lax.fori_loop(0, SB // BC, batch_body, 0)
    pltpu.make_async_copy(rout, acc.at[dloc], sem3).wait()
    plsc.subcore_barrier()
    pltpu.sync_copy(acc.at[pl.ds(s * TPH, TPH)],
                    nd2_hbm.at[pl.ds(lo + s * TPH, TPH)])


def _tc3_body(nd2_ref, out_ref):
    nd = nd2_ref[...]
    den = nd[:, 3:4]
    o = [jnp.maximum(nd[:, j:j + 1] / (den + 1e-16), 0.0) for j in range(3)]
    om = jnp.maximum(jnp.maximum(o[0], o[1]), o[2])
    es = jnp.exp(o[0] - om) + jnp.exp(o[1] - om) + jnp.exp(o[2] - om)
    lse = jnp.log(es) + om
    cols = [o[j] - lse for j in range(3)]
    cols.append(jnp.zeros((R2, 13), jnp.float32))
    out_ref[...] = jnp.concatenate(cols, axis=1)


_tc3 = pl.pallas_call(
    _tc3_body,
    grid=(NP // R2,),
    in_specs=[pl.BlockSpec((R2, 16), lambda i: (i, 0))],
    out_specs=pl.BlockSpec((R2, 16), lambda i: (i, 0)),
    out_shape=jax.ShapeDtypeStruct((NP, 16), jnp.float32),
)


_SELS = np.zeros((C, 16), np.float32)
_SELD = np.zeros((C, 16), np.float32)
for _h in range(H):
    for _d in range(D):
        _SELS[_h * D + _d, _h] = 1.0
        _SELD[_h * D + _d, 4 + _h] = 1.0


def kernel(x_patient, x_symptom, x_dosha, ei_has_trait, ei_belongs_to,
           ei_similar_to, p1_proj_patient_w, p1_proj_patient_b,
           p1_proj_symptom_w, p1_proj_symptom_b, p1_proj_dosha_w,
           p1_proj_dosha_b, p1_src_has_trait, p1_dst_has_trait,
           p1_src_belongs_to, p1_dst_belongs_to, p1_src_similar_to,
           p1_dst_similar_to, p1_k_w, p1_k_b, p1_q, p2_proj_patient_w,
           p2_proj_patient_b, p2_proj_symptom_w, p2_proj_symptom_b,
           p2_proj_dosha_w, p2_proj_dosha_b, p2_src_has_trait,
           p2_dst_has_trait, p2_src_belongs_to, p2_dst_belongs_to,
           p2_src_similar_to, p2_dst_similar_to, p2_k_w, p2_k_b, p2_q,
           bn_w, bn_b, bn_rm, bn_rv):
    src = ei_similar_to[0]
    dst = ei_similar_to[1]
    pad = jnp.zeros((EP - E,), jnp.int32)
    srcp = jnp.concatenate([src, pad])
    dstp = jnp.concatenate([dst, pad])

    satt = p1_src_similar_to.reshape(1, C)
    datt = p1_dst_similar_to.reshape(1, C)
    xp, asd, mx = _tc1(x_patient, p1_proj_patient_w,
                       p1_proj_patient_b.reshape(1, C), satt, datt,
                       jnp.asarray(_SELS), jnp.asarray(_SELD))

    e_t = _sc_a(asd, srcp, dstp, mx)
    xp2d = xp.reshape(2 * N, XC)
    nd = _sc_b(xp2d, dstp, e_t)

    bns = (bn_w / jnp.sqrt(bn_rv + 1e-5)).reshape(1, C)
    bnb = (bn_b - bn_rm * bns[0]).reshape(1, C)
    s2 = p2_src_similar_to.reshape(3)
    d2 = p2_dst_similar_to.reshape(3)
    pmat = jnp.zeros((C, 16), jnp.float32)
    pmat = pmat.at[:, 0:3].set(p2_proj_patient_w)
    pmat = pmat.at[:, 3].set(p2_proj_patient_w @ s2)
    pmat = pmat.at[:, 4].set(p2_proj_patient_w @ d2)
    pb = jnp.zeros((16,), jnp.float32)
    pb = pb.at[0:3].set(p2_proj_patient_b)
    pb = pb.at[3].set(p2_proj_patient_b @ s2)
    pb = pb.at[4].set(p2_proj_patient_b @ d2)

    pk, mx2 = _tc2(nd, bns, bnb, pmat, pb.reshape(1, 16))
    nd2 = _sc_c(pk, srcp, dstp, mx2)
    out = _tc3(nd2)
    return out[:N, :3]
